# Initial kernel scaffold; baseline (speedup 1.0000x reference)
import jax
import jax.numpy as jnp
from jax.experimental import pallas as pl


def kernel(x, W1, b1, W2, b2, W3, b3, W4, b4):
    raise NotImplementedError("write your pallas kernel here")


# speedup vs baseline: 40.1957x; 40.1957x over previous
import functools

import jax
import jax.numpy as jnp
from jax.experimental import pallas as pl

SEQ = 2048
ENC = 512
HID = 1024
E = 8
B = 8192

TB = 256


def _mlp_kernel(ids_ref, x_ref, w1_ref, b1_ref, w2_ref, b2_ref,
                w3_ref, b3_ref, w4_ref, b4_ref, out_ref):
    e = pl.program_id(1)
    xv = x_ref[...]
    h = jnp.maximum(
        jnp.dot(xv, w1_ref[0], preferred_element_type=jnp.float32) + b1_ref[0], 0.0)
    z = jnp.maximum(
        jnp.dot(h, w2_ref[0], preferred_element_type=jnp.float32) + b2_ref[0], 0.0)
    h2 = jnp.maximum(
        jnp.dot(z, w3_ref[0], preferred_element_type=jnp.float32) + b3_ref[0], 0.0)
    y = jnp.dot(h2, w4_ref[0], preferred_element_type=jnp.float32) + b4_ref[0]
    mask = ids_ref[...] == e

    @pl.when(e == 0)
    def _():
        out_ref[...] = jnp.where(mask, y, 0.0)

    @pl.when(e > 0)
    def _():
        out_ref[...] = jnp.where(mask, y, out_ref[...])


@jax.jit
def kernel(x, W1, b1, W2, b2, W3, b3, W4, b4):
    feats = x[:, :SEQ]
    ids = x[:, SEQ:].astype(jnp.int32)
    T = B // TB
    grid = (T, E)
    out = pl.pallas_call(
        _mlp_kernel,
        grid=grid,
        in_specs=[
            pl.BlockSpec((TB, 1), lambda t, e: (t, 0)),
            pl.BlockSpec((TB, SEQ), lambda t, e: (t, 0)),
            pl.BlockSpec((1, SEQ, HID), lambda t, e: (e, 0, 0)),
            pl.BlockSpec((1, 1, HID), lambda t, e: (e, 0, 0)),
            pl.BlockSpec((1, HID, ENC), lambda t, e: (e, 0, 0)),
            pl.BlockSpec((1, 1, ENC), lambda t, e: (e, 0, 0)),
            pl.BlockSpec((1, ENC, HID), lambda t, e: (e, 0, 0)),
            pl.BlockSpec((1, 1, HID), lambda t, e: (e, 0, 0)),
            pl.BlockSpec((1, HID, SEQ), lambda t, e: (e, 0, 0)),
            pl.BlockSpec((1, 1, SEQ), lambda t, e: (e, 0, 0)),
        ],
        out_specs=pl.BlockSpec((TB, SEQ), lambda t, e: (t, 0)),
        out_shape=jax.ShapeDtypeStruct((B, SEQ), jnp.float32),
    )(ids, feats, W1, b1.reshape(E, 1, HID), W2, b2.reshape(E, 1, ENC),
      W3, b3.reshape(E, 1, HID), W4, b4.reshape(E, 1, SEQ))
    return out


# speedup vs baseline: 95.1169x; 2.3663x over previous
import functools

import jax
import jax.numpy as jnp
from jax.experimental import pallas as pl
from jax.experimental.pallas import tpu as pltpu
from jax.experimental.pallas import tpu_sc as plsc

SEQ = 2048
ENC = 512
HID = 1024
E = 8
B = 8192

TBG = 256
T = B // TBG
WMAX = T + E - 1
CHUNK = 8
CW = SEQ // CHUNK
NR = B * CHUNK
SC_WIN = 128


@functools.cache
def _vector_mesh():
    return plsc.VectorSubcoreMesh(
        core_axis_name="core", subcore_axis_name="subcore")


TBC = 256


def _routing_kernel(ids_ref, dest4_ref, off_ref, oh_ref, cs_ref):
    ids = ids_ref[...]
    lane = jax.lax.broadcasted_iota(jnp.int32, (B, E), 1)
    oh = (ids == lane).astype(jnp.float32)
    oh_ref[...] = oh
    r = jax.lax.broadcasted_iota(jnp.int32, (TBC, TBC), 0)
    c = jax.lax.broadcasted_iota(jnp.int32, (TBC, TBC), 1)
    ltri = (c <= r).astype(jnp.float32)

    def body(ci, carry):
        chunk = oh_ref[pl.ds(ci * TBC, TBC), :]
        cs_ref[pl.ds(ci * TBC, TBC), :] = (
            jnp.dot(ltri, chunk, preferred_element_type=jnp.float32) + carry)
        return carry + jnp.sum(chunk, axis=0, keepdims=True)

    counts = jax.lax.fori_loop(0, B // TBC, body,
                               jnp.zeros((1, E), jnp.float32))
    csum = cs_ref[...]
    rank = jnp.sum(oh * csum, axis=1, keepdims=True) - 1.0
    re = jax.lax.broadcasted_iota(jnp.int32, (E, E), 0)
    ce = jax.lax.broadcasted_iota(jnp.int32, (E, E), 1)
    m = (re < ce).astype(jnp.float32)
    offs = jnp.dot(counts, m, preferred_element_type=jnp.float32,
                   precision=jax.lax.Precision.HIGHEST)
    dest = rank + jnp.sum(oh * offs, axis=1, keepdims=True)
    k = jax.lax.broadcasted_iota(jnp.int32, (B, CHUNK), 1)
    dest4_ref[...] = CHUNK * dest.astype(jnp.int32) + k
    off_ref[...] = offs.astype(jnp.int32)


def _routing(ids):
    return pl.pallas_call(
        _routing_kernel,
        out_shape=(jax.ShapeDtypeStruct((B, CHUNK), jnp.int32),
                   jax.ShapeDtypeStruct((1, E), jnp.int32)),
        scratch_shapes=[pltpu.VMEM((B, E), jnp.float32),
                        pltpu.VMEM((B, E), jnp.float32)],
    )(ids)


def _dispatch(feats4, idx):
    @functools.partial(
        pl.kernel,
        out_type=jax.ShapeDtypeStruct((NR, CW), feats4.dtype),
        mesh=_vector_mesh())
    def run(x_hbm, i_hbm, o_hbm):
        def body(x_vmem, i_vmem):
            pltpu.sync_copy(x_vmem, o_hbm.at[i_vmem.at[0]])

        pltpu.emit_pipeline(
            body,
            grid=(NR // SC_WIN,),
            in_specs=[pl.BlockSpec((SC_WIN, CW), lambda i: (i, 0)),
                      pl.BlockSpec((1, SC_WIN), lambda i: (0, i))],
            out_specs=[],
            core_axis_name=("core", "subcore"),
            dimension_semantics=(pltpu.PARALLEL,),
        )(x_hbm, i_hbm)

    return run(feats4, idx)


def _combine(ys4, idx):
    @functools.partial(
        pl.kernel,
        out_type=jax.ShapeDtypeStruct((NR, CW), ys4.dtype),
        mesh=_vector_mesh())
    def run(ys_hbm, i_hbm, o_hbm):
        def body(i_vmem, o_vmem):
            pltpu.sync_copy(ys_hbm.at[i_vmem.at[0]], o_vmem)

        pltpu.emit_pipeline(
            body,
            grid=(NR // SC_WIN,),
            in_specs=[pl.BlockSpec((1, SC_WIN), lambda i: (0, i))],
            out_specs=[pl.BlockSpec((SC_WIN, CW), lambda i: (i, 0))],
            core_axis_name=("core", "subcore"),
            dimension_semantics=(pltpu.PARALLEL,),
        )(i_hbm, o_hbm)

    return run(ys4, idx)


def _metadata(offsets):
    o = offsets
    ends = jnp.concatenate([o[1:], jnp.full((1,), B, jnp.int32)])
    counts = ends - o
    f = o // TBG
    l = (ends - 1) // TBG
    tpg = jnp.where(counts > 0, l - f + 1, 0)
    cw = jnp.concatenate([jnp.zeros((1,), jnp.int32),
                          jnp.cumsum(tpg).astype(jnp.int32)])
    total = cw[E]
    w = jnp.arange(WMAX, dtype=jnp.int32)
    gid = jnp.sum((w[:, None] >= cw[None, 1:]).astype(jnp.int32), axis=1)
    gid = jnp.minimum(gid, E - 1)
    tile = f[gid] + (w - cw[gid])
    valid = w < total
    tile = jnp.where(valid, tile, T - 1)
    start = jnp.where(valid, jnp.maximum(o[gid], tile * TBG), 0)
    end = jnp.where(valid, jnp.minimum(ends[gid], (tile + 1) * TBG), 0)
    first = jnp.concatenate([jnp.ones((1,), jnp.int32),
                             (tile[1:] != tile[:-1]).astype(jnp.int32)])
    return tile, gid, start, end, first


def _mlp_kernel(tl_ref, gd_ref, st_ref, en_ref, fr_ref,
                xs_ref, w1_ref, b1_ref, w2_ref, b2_ref,
                w3_ref, b3_ref, w4_ref, b4_ref, out_ref):
    w = pl.program_id(0)
    start, end, first = st_ref[w], en_ref[w], fr_ref[w]

    @pl.when(start < end)
    def _():
        xv = xs_ref[...].astype(jnp.bfloat16)
        h = jnp.maximum(
            jnp.dot(xv, w1_ref[0], preferred_element_type=jnp.float32)
            + b1_ref[0], 0.0).astype(jnp.bfloat16)
        z = jnp.maximum(
            jnp.dot(h, w2_ref[0], preferred_element_type=jnp.float32)
            + b2_ref[0], 0.0).astype(jnp.bfloat16)
        h2 = jnp.maximum(
            jnp.dot(z, w3_ref[0], preferred_element_type=jnp.float32)
            + b3_ref[0], 0.0).astype(jnp.bfloat16)
        y = (jnp.dot(h2, w4_ref[0], preferred_element_type=jnp.float32)
             + b4_ref[0])
        rows = (tl_ref[w] * TBG
                + jax.lax.broadcasted_iota(jnp.int32, (TBG, 1), 0))
        m = (rows >= start) & (rows < end)

        @pl.when(first == 1)
        def _():
            out_ref[...] = jnp.where(m, y, 0.0)

        @pl.when(first == 0)
        def _():
            out_ref[...] = jnp.where(m, y, out_ref[...])


def _grouped_mlp(xs, meta, W1, b1, W2, b2, W3, b3, W4, b4):
    tile, gid, start, end, first = meta
    grid_spec = pltpu.PrefetchScalarGridSpec(
        num_scalar_prefetch=5,
        grid=(WMAX,),
        in_specs=[
            pl.BlockSpec((TBG, SEQ), lambda w, tl, gd, st, en, fr: (tl[w], 0)),
            pl.BlockSpec((1, SEQ, HID),
                         lambda w, tl, gd, st, en, fr: (gd[w], 0, 0)),
            pl.BlockSpec((1, 1, HID),
                         lambda w, tl, gd, st, en, fr: (gd[w], 0, 0)),
            pl.BlockSpec((1, HID, ENC),
                         lambda w, tl, gd, st, en, fr: (gd[w], 0, 0)),
            pl.BlockSpec((1, 1, ENC),
                         lambda w, tl, gd, st, en, fr: (gd[w], 0, 0)),
            pl.BlockSpec((1, ENC, HID),
                         lambda w, tl, gd, st, en, fr: (gd[w], 0, 0)),
            pl.BlockSpec((1, 1, HID),
                         lambda w, tl, gd, st, en, fr: (gd[w], 0, 0)),
            pl.BlockSpec((1, HID, SEQ),
                         lambda w, tl, gd, st, en, fr: (gd[w], 0, 0)),
            pl.BlockSpec((1, 1, SEQ),
                         lambda w, tl, gd, st, en, fr: (gd[w], 0, 0)),
        ],
        out_specs=pl.BlockSpec((TBG, SEQ),
                               lambda w, tl, gd, st, en, fr: (tl[w], 0)),
    )
    return pl.pallas_call(
        _mlp_kernel,
        grid_spec=grid_spec,
        out_shape=jax.ShapeDtypeStruct((B, SEQ), jnp.float32),
    )(tile, gid, start, end, first, xs,
      W1, b1.reshape(E, 1, HID), W2, b2.reshape(E, 1, ENC),
      W3, b3.reshape(E, 1, HID), W4, b4.reshape(E, 1, SEQ))


@jax.jit
def kernel(x, W1, b1, W2, b2, W3, b3, W4, b4):
    feats4 = x[:, :SEQ].reshape(NR, CW)
    ids = x[:, SEQ].astype(jnp.int32).reshape(B, 1)
    dest4, off = _routing(ids)
    idx = dest4.reshape(1, NR)
    meta = _metadata(off[0])
    xs = _dispatch(feats4, idx).reshape(B, SEQ)
    ys = _grouped_mlp(xs, meta,
                      W1.astype(jnp.bfloat16), b1, W2.astype(jnp.bfloat16),
                      b2, W3.astype(jnp.bfloat16), b3,
                      W4.astype(jnp.bfloat16), b4)
    y = _combine(ys.reshape(NR, CW), idx).reshape(B, SEQ)
    return y


# speedup vs baseline: 95.3261x; 1.0022x over previous
import functools

import jax
import jax.numpy as jnp
from jax.experimental import pallas as pl
from jax.experimental.pallas import tpu as pltpu
from jax.experimental.pallas import tpu_sc as plsc

SEQ = 2048
ENC = 512
HID = 1024
E = 8
B = 8192

TBG = 256
T = B // TBG
WMAX = T + E - 1
CHUNK = 8
CW = SEQ // CHUNK
NR = B * CHUNK
SC_WIN = 128


@functools.cache
def _vector_mesh():
    return plsc.VectorSubcoreMesh(
        core_axis_name="core", subcore_axis_name="subcore")


TBC = 256


NC = B // TBC


def _routing_kernel(ids_ref, dest4_ref, off_ref, oh_ref, cs_ref, tot_ref):
    ids = ids_ref[...]
    lane = jax.lax.broadcasted_iota(jnp.int32, (B, E), 1)
    oh = (ids == lane).astype(jnp.float32)
    oh_ref[...] = oh
    r = jax.lax.broadcasted_iota(jnp.int32, (TBC, TBC), 0)
    c = jax.lax.broadcasted_iota(jnp.int32, (TBC, TBC), 1)
    ltri = (c <= r).astype(jnp.float32)

    for ci in range(NC):
        chunk = oh_ref[pl.ds(ci * TBC, TBC), :]
        cs = jnp.dot(ltri, chunk, preferred_element_type=jnp.float32)
        cs_ref[pl.ds(ci * TBC, TBC), :] = cs
        tot_ref[ci, :] = cs[TBC - 1, :]

    rc = jax.lax.broadcasted_iota(jnp.int32, (NC, NC), 0)
    cc = jax.lax.broadcasted_iota(jnp.int32, (NC, NC), 1)
    strict = (cc < rc).astype(jnp.float32)
    tot = tot_ref[...]
    carry = jnp.dot(strict, tot, preferred_element_type=jnp.float32)
    counts = jnp.sum(tot, axis=0, keepdims=True)

    rowc = jax.lax.broadcasted_iota(jnp.int32, (B, NC), 0) // TBC
    sel = (rowc == jax.lax.broadcasted_iota(jnp.int32, (B, NC), 1)
           ).astype(jnp.float32)
    carry_rows = jnp.dot(sel, carry, preferred_element_type=jnp.float32,
                         precision=jax.lax.Precision.HIGHEST)
    csum = cs_ref[...] + carry_rows
    rank = jnp.sum(oh * csum, axis=1, keepdims=True) - 1.0
    re = jax.lax.broadcasted_iota(jnp.int32, (E, E), 0)
    ce = jax.lax.broadcasted_iota(jnp.int32, (E, E), 1)
    m = (re < ce).astype(jnp.float32)
    offs = jnp.dot(counts, m, preferred_element_type=jnp.float32,
                   precision=jax.lax.Precision.HIGHEST)
    dest = rank + jnp.sum(oh * offs, axis=1, keepdims=True)
    k = jax.lax.broadcasted_iota(jnp.int32, (B, CHUNK), 1)
    dest4_ref[...] = CHUNK * dest.astype(jnp.int32) + k
    off_ref[...] = offs.astype(jnp.int32)


def _routing(ids):
    return pl.pallas_call(
        _routing_kernel,
        out_shape=(jax.ShapeDtypeStruct((B, CHUNK), jnp.int32),
                   jax.ShapeDtypeStruct((1, E), jnp.int32)),
        scratch_shapes=[pltpu.VMEM((B, E), jnp.float32),
                        pltpu.VMEM((B, E), jnp.float32),
                        pltpu.VMEM((NC, E), jnp.float32)],
    )(ids)


def _dispatch(feats4, idx):
    @functools.partial(
        pl.kernel,
        out_type=jax.ShapeDtypeStruct((NR, CW), feats4.dtype),
        mesh=_vector_mesh())
    def run(x_hbm, i_hbm, o_hbm):
        def body(x_vmem, i_vmem):
            pltpu.sync_copy(x_vmem, o_hbm.at[i_vmem.at[0]])

        pltpu.emit_pipeline(
            body,
            grid=(NR // SC_WIN,),
            in_specs=[pl.BlockSpec((SC_WIN, CW), lambda i: (i, 0)),
                      pl.BlockSpec((1, SC_WIN), lambda i: (0, i))],
            out_specs=[],
            core_axis_name=("core", "subcore"),
            dimension_semantics=(pltpu.PARALLEL,),
        )(x_hbm, i_hbm)

    return run(feats4, idx)


def _combine(ys4, idx):
    @functools.partial(
        pl.kernel,
        out_type=jax.ShapeDtypeStruct((NR, CW), ys4.dtype),
        mesh=_vector_mesh())
    def run(ys_hbm, i_hbm, o_hbm):
        def body(i_vmem, o_vmem):
            pltpu.sync_copy(ys_hbm.at[i_vmem.at[0]], o_vmem)

        pltpu.emit_pipeline(
            body,
            grid=(NR // SC_WIN,),
            in_specs=[pl.BlockSpec((1, SC_WIN), lambda i: (0, i))],
            out_specs=[pl.BlockSpec((SC_WIN, CW), lambda i: (i, 0))],
            core_axis_name=("core", "subcore"),
            dimension_semantics=(pltpu.PARALLEL,),
        )(i_hbm, o_hbm)

    return run(ys4, idx)


def _metadata(offsets):
    o = offsets
    ends = jnp.concatenate([o[1:], jnp.full((1,), B, jnp.int32)])
    counts = ends - o
    f = o // TBG
    l = (ends - 1) // TBG
    tpg = jnp.where(counts > 0, l - f + 1, 0)
    cw = jnp.concatenate([jnp.zeros((1,), jnp.int32),
                          jnp.cumsum(tpg).astype(jnp.int32)])
    total = cw[E]
    w = jnp.arange(WMAX, dtype=jnp.int32)
    gid = jnp.sum((w[:, None] >= cw[None, 1:]).astype(jnp.int32), axis=1)
    gid = jnp.minimum(gid, E - 1)
    tile = f[gid] + (w - cw[gid])
    valid = w < total
    tile = jnp.where(valid, tile, T - 1)
    start = jnp.where(valid, jnp.maximum(o[gid], tile * TBG), 0)
    end = jnp.where(valid, jnp.minimum(ends[gid], (tile + 1) * TBG), 0)
    first = (start == tile * TBG).astype(jnp.int32)
    return tile, gid, start, end, first


def _mlp_kernel(tl_ref, gd_ref, st_ref, en_ref, fr_ref,
                xs_ref, w1_ref, b1_ref, w2_ref, b2_ref,
                w3_ref, b3_ref, w4_ref, b4_ref, out_ref):
    w = pl.program_id(0)
    start, end, first = st_ref[w], en_ref[w], fr_ref[w]

    @pl.when(start < end)
    def _():
        xv = xs_ref[...].astype(jnp.bfloat16)
        h = jnp.maximum(
            jnp.dot(xv, w1_ref[0], preferred_element_type=jnp.float32)
            + b1_ref[0], 0.0).astype(jnp.bfloat16)
        z = jnp.maximum(
            jnp.dot(h, w2_ref[0], preferred_element_type=jnp.float32)
            + b2_ref[0], 0.0).astype(jnp.bfloat16)
        h2 = jnp.maximum(
            jnp.dot(z, w3_ref[0], preferred_element_type=jnp.float32)
            + b3_ref[0], 0.0).astype(jnp.bfloat16)
        y = (jnp.dot(h2, w4_ref[0], preferred_element_type=jnp.float32)
             + b4_ref[0])
        rows = (tl_ref[w] * TBG
                + jax.lax.broadcasted_iota(jnp.int32, (TBG, 1), 0))
        m = (rows >= start) & (rows < end)

        @pl.when(first == 1)
        def _():
            out_ref[...] = jnp.where(m, y, 0.0)

        @pl.when(first == 0)
        def _():
            out_ref[...] = jnp.where(m, y, out_ref[...])


def _grouped_mlp(xs, meta, W1, b1, W2, b2, W3, b3, W4, b4):
    tile, gid, start, end, first = meta
    grid_spec = pltpu.PrefetchScalarGridSpec(
        num_scalar_prefetch=5,
        grid=(WMAX,),
        in_specs=[
            pl.BlockSpec((TBG, SEQ), lambda w, tl, gd, st, en, fr: (tl[w], 0)),
            pl.BlockSpec((1, SEQ, HID),
                         lambda w, tl, gd, st, en, fr: (gd[w], 0, 0)),
            pl.BlockSpec((1, 1, HID),
                         lambda w, tl, gd, st, en, fr: (gd[w], 0, 0)),
            pl.BlockSpec((1, HID, ENC),
                         lambda w, tl, gd, st, en, fr: (gd[w], 0, 0)),
            pl.BlockSpec((1, 1, ENC),
                         lambda w, tl, gd, st, en, fr: (gd[w], 0, 0)),
            pl.BlockSpec((1, ENC, HID),
                         lambda w, tl, gd, st, en, fr: (gd[w], 0, 0)),
            pl.BlockSpec((1, 1, HID),
                         lambda w, tl, gd, st, en, fr: (gd[w], 0, 0)),
            pl.BlockSpec((1, HID, SEQ),
                         lambda w, tl, gd, st, en, fr: (gd[w], 0, 0)),
            pl.BlockSpec((1, 1, SEQ),
                         lambda w, tl, gd, st, en, fr: (gd[w], 0, 0)),
        ],
        out_specs=pl.BlockSpec((TBG, SEQ),
                               lambda w, tl, gd, st, en, fr: (tl[w], 0)),
    )
    return pl.pallas_call(
        _mlp_kernel,
        grid_spec=grid_spec,
        out_shape=jax.ShapeDtypeStruct((B, SEQ), jnp.float32),
    )(tile, gid, start, end, first, xs,
      W1, b1.reshape(E, 1, HID), W2, b2.reshape(E, 1, ENC),
      W3, b3.reshape(E, 1, HID), W4, b4.reshape(E, 1, SEQ))


@jax.jit
def kernel(x, W1, b1, W2, b2, W3, b3, W4, b4):
    feats4 = x[:, :SEQ].reshape(NR, CW)
    ids = x[:, SEQ].astype(jnp.int32).reshape(B, 1)
    dest4, off = _routing(ids)
    idx = dest4.reshape(1, NR)
    meta = _metadata(off[0])
    xs = _dispatch(feats4, idx).reshape(B, SEQ)
    ys = _grouped_mlp(xs, meta,
                      W1.astype(jnp.bfloat16), b1, W2.astype(jnp.bfloat16),
                      b2, W3.astype(jnp.bfloat16), b3,
                      W4.astype(jnp.bfloat16), b4)
    y = _combine(ys.reshape(NR, CW), idx).reshape(B, SEQ)
    return y


# speedup vs baseline: 133.7410x; 1.4030x over previous
import functools

import jax
import jax.numpy as jnp
from jax.experimental import pallas as pl
from jax.experimental.pallas import tpu as pltpu
from jax.experimental.pallas import tpu_sc as plsc

SEQ = 2048
ENC = 512
HID = 1024
E = 8
B = 8192

TBG = 256
T = B // TBG
WMAX = T + E - 1
CHUNK = 8
CW = SEQ // CHUNK
NR = B * CHUNK
SC_WIN = 128
RW = SC_WIN // CHUNK


@functools.cache
def _vector_mesh():
    return plsc.VectorSubcoreMesh(
        core_axis_name="core", subcore_axis_name="subcore")


TBC = 256
NC = B // TBC


def _routing_kernel(ids_ref, dest_ref, off_ref, oh_ref, cs_ref, tot_ref):
    ids = ids_ref[...]
    lane = jax.lax.broadcasted_iota(jnp.int32, (B, E), 1)
    oh = (ids == lane).astype(jnp.float32)
    oh_ref[...] = oh
    r = jax.lax.broadcasted_iota(jnp.int32, (TBC, TBC), 0)
    c = jax.lax.broadcasted_iota(jnp.int32, (TBC, TBC), 1)
    ltri = (c <= r).astype(jnp.float32)

    for ci in range(NC):
        chunk = oh_ref[pl.ds(ci * TBC, TBC), :]
        cs = jnp.dot(ltri, chunk, preferred_element_type=jnp.float32)
        cs_ref[pl.ds(ci * TBC, TBC), :] = cs
        tot_ref[ci, :] = cs[TBC - 1, :]

    rc = jax.lax.broadcasted_iota(jnp.int32, (NC, NC), 0)
    cc = jax.lax.broadcasted_iota(jnp.int32, (NC, NC), 1)
    strict = (cc < rc).astype(jnp.float32)
    tot = tot_ref[...]
    carry = jnp.dot(strict, tot, preferred_element_type=jnp.float32)
    counts = jnp.sum(tot, axis=0, keepdims=True)

    rowc = jax.lax.broadcasted_iota(jnp.int32, (B, NC), 0) // TBC
    sel = (rowc == jax.lax.broadcasted_iota(jnp.int32, (B, NC), 1)
           ).astype(jnp.float32)
    carry_rows = jnp.dot(sel, carry, preferred_element_type=jnp.float32,
                         precision=jax.lax.Precision.HIGHEST)
    csum = cs_ref[...] + carry_rows
    rank = jnp.sum(oh * csum, axis=1, keepdims=True) - 1.0
    re = jax.lax.broadcasted_iota(jnp.int32, (E, E), 0)
    ce = jax.lax.broadcasted_iota(jnp.int32, (E, E), 1)
    m = (re < ce).astype(jnp.float32)
    offs = jnp.dot(counts, m, preferred_element_type=jnp.float32,
                   precision=jax.lax.Precision.HIGHEST)
    dest = rank + jnp.sum(oh * offs, axis=1, keepdims=True)
    dest_ref[...] = dest.astype(jnp.int32)
    off_ref[...] = offs.astype(jnp.int32)


def _routing(ids):
    return pl.pallas_call(
        _routing_kernel,
        out_shape=(jax.ShapeDtypeStruct((B, 1), jnp.int32),
                   jax.ShapeDtypeStruct((1, E), jnp.int32)),
        scratch_shapes=[pltpu.VMEM((B, E), jnp.float32),
                        pltpu.VMEM((B, E), jnp.float32),
                        pltpu.VMEM((NC, E), jnp.float32)],
    )(ids)


def _dispatch(x, idx):
    @functools.partial(
        pl.kernel,
        out_type=jax.ShapeDtypeStruct((NR, CW), jnp.float32),
        mesh=_vector_mesh())
    def run(x_hbm, i_hbm, o_hbm):
        def body(x_vmem, i_vmem):
            for k in range(CHUNK):
                pltpu.sync_copy(
                    x_vmem.at[:, pl.ds(k * CW, CW)],
                    o_hbm.at[i_vmem.at[0, pl.ds(k * RW, RW)]])

        pltpu.emit_pipeline(
            body,
            grid=(B // RW,),
            in_specs=[pl.BlockSpec((RW, SEQ + 1), lambda i: (i, 0)),
                      pl.BlockSpec((1, SC_WIN), lambda i: (0, i))],
            out_specs=[],
            core_axis_name=("core", "subcore"),
            dimension_semantics=(pltpu.PARALLEL,),
        )(x_hbm, i_hbm)

    return run(x, idx)


def _combine(ys4, idx):
    @functools.partial(
        pl.kernel,
        out_type=jax.ShapeDtypeStruct((B, SEQ), jnp.float32),
        mesh=_vector_mesh())
    def run(ys_hbm, i_hbm, o_hbm):
        def body(i_vmem, o_vmem):
            for k in range(CHUNK):
                pltpu.sync_copy(
                    ys_hbm.at[i_vmem.at[0, pl.ds(k * RW, RW)]],
                    o_vmem.at[:, pl.ds(k * CW, CW)])

        pltpu.emit_pipeline(
            body,
            grid=(B // RW,),
            in_specs=[pl.BlockSpec((1, SC_WIN), lambda i: (0, i))],
            out_specs=[pl.BlockSpec((RW, SEQ), lambda i: (i, 0))],
            core_axis_name=("core", "subcore"),
            dimension_semantics=(pltpu.PARALLEL,),
        )(i_hbm, o_hbm)

    return run(ys4, idx)


def _metadata(offsets):
    o = offsets
    ends = jnp.concatenate([o[1:], jnp.full((1,), B, jnp.int32)])
    counts = ends - o
    f = o // TBG
    l = (ends - 1) // TBG
    tpg = jnp.where(counts > 0, l - f + 1, 0)
    cw = jnp.concatenate([jnp.zeros((1,), jnp.int32),
                          jnp.cumsum(tpg).astype(jnp.int32)])
    total = cw[E]
    w = jnp.arange(WMAX, dtype=jnp.int32)
    gid = jnp.sum((w[:, None] >= cw[None, 1:]).astype(jnp.int32), axis=1)
    gid = jnp.minimum(gid, E - 1)
    tile = f[gid] + (w - cw[gid])
    valid = w < total
    tile = jnp.where(valid, tile, T - 1)
    start = jnp.where(valid, jnp.maximum(o[gid], tile * TBG), 0)
    end = jnp.where(valid, jnp.minimum(ends[gid], (tile + 1) * TBG), 0)
    first = (start == tile * TBG).astype(jnp.int32)
    return tile, gid, start, end, first


def _mlp_kernel(tl_ref, gd_ref, st_ref, en_ref, fr_ref,
                xs_ref, w1_ref, b1_ref, w2_ref, b2_ref,
                w3_ref, b3_ref, w4_ref, b4_ref, out_ref):
    w = pl.program_id(0)
    start, end, first = st_ref[w], en_ref[w], fr_ref[w]

    @pl.when(start < end)
    def _():
        acc = jnp.zeros((TBG, HID), jnp.float32)
        for k in range(CHUNK):
            acc = acc + jnp.dot(xs_ref[k],
                                w1_ref[0, pl.ds(k * CW, CW), :],
                                preferred_element_type=jnp.float32)
        h = jnp.maximum(acc + b1_ref[0], 0.0)
        z = jnp.maximum(
            jnp.dot(h, w2_ref[0], preferred_element_type=jnp.float32)
            + b2_ref[0], 0.0)
        h2 = jnp.maximum(
            jnp.dot(z, w3_ref[0], preferred_element_type=jnp.float32)
            + b3_ref[0], 0.0)
        rows = (tl_ref[w] * TBG
                + jax.lax.broadcasted_iota(jnp.int32, (TBG, 1), 0))
        m = (rows >= start) & (rows < end)
        for k in range(CHUNK):
            yk = (jnp.dot(h2, w4_ref[0, :, pl.ds(k * CW, CW)],
                          preferred_element_type=jnp.float32)
                  + b4_ref[0, :, pl.ds(k * CW, CW)])

            @pl.when(first == 1)
            def _(yk=yk, k=k):
                out_ref[k] = jnp.where(m, yk, 0.0)

            @pl.when(first == 0)
            def _(yk=yk, k=k):
                out_ref[k] = jnp.where(m, yk, out_ref[k])


def _grouped_mlp(xs5, meta, W1, b1, W2, b2, W3, b3, W4, b4):
    tile, gid, start, end, first = meta
    grid_spec = pltpu.PrefetchScalarGridSpec(
        num_scalar_prefetch=5,
        grid=(WMAX,),
        in_specs=[
            pl.BlockSpec((CHUNK, TBG, CW),
                         lambda w, tl, gd, st, en, fr: (0, tl[w], 0)),
            pl.BlockSpec((1, SEQ, HID),
                         lambda w, tl, gd, st, en, fr: (gd[w], 0, 0)),
            pl.BlockSpec((1, 1, HID),
                         lambda w, tl, gd, st, en, fr: (gd[w], 0, 0)),
            pl.BlockSpec((1, HID, ENC),
                         lambda w, tl, gd, st, en, fr: (gd[w], 0, 0)),
            pl.BlockSpec((1, 1, ENC),
                         lambda w, tl, gd, st, en, fr: (gd[w], 0, 0)),
            pl.BlockSpec((1, ENC, HID),
                         lambda w, tl, gd, st, en, fr: (gd[w], 0, 0)),
            pl.BlockSpec((1, 1, HID),
                         lambda w, tl, gd, st, en, fr: (gd[w], 0, 0)),
            pl.BlockSpec((1, HID, SEQ),
                         lambda w, tl, gd, st, en, fr: (gd[w], 0, 0)),
            pl.BlockSpec((1, 1, SEQ),
                         lambda w, tl, gd, st, en, fr: (gd[w], 0, 0)),
        ],
        out_specs=pl.BlockSpec((CHUNK, TBG, CW),
                               lambda w, tl, gd, st, en, fr: (0, tl[w], 0)),
    )
    return pl.pallas_call(
        _mlp_kernel,
        grid_spec=grid_spec,
        out_shape=jax.ShapeDtypeStruct((CHUNK, B, CW), jnp.float32),
    )(tile, gid, start, end, first, xs5,
      W1, b1.reshape(E, 1, HID), W2, b2.reshape(E, 1, ENC),
      W3, b3.reshape(E, 1, HID), W4, b4.reshape(E, 1, SEQ))


@jax.jit
def kernel(x, W1, b1, W2, b2, W3, b3, W4, b4):
    ids = x[:, SEQ].astype(jnp.int32).reshape(B, 1)
    dest, off = _routing(ids)
    dest_r = dest.reshape(B // RW, RW)
    idx = (dest_r[:, None, :]
           + (B * jnp.arange(CHUNK, dtype=jnp.int32))[None, :, None])
    idx = idx.reshape(1, NR)
    meta = _metadata(off[0])
    xs4 = _dispatch(x, idx)
    xs5 = xs4.reshape(CHUNK, B, CW)
    ys5 = _grouped_mlp(xs5, meta, W1, b1, W2, b2, W3, b3, W4, b4)
    ys4 = ys5.reshape(NR, CW)
    return _combine(ys4, idx)


# speedup vs baseline: 155.8626x; 1.1654x over previous
import functools

import jax
import jax.numpy as jnp
from jax.experimental import pallas as pl
from jax.experimental.pallas import tpu as pltpu
from jax.experimental.pallas import tpu_sc as plsc

SEQ = 2048
ENC = 512
HID = 1024
E = 8
B = 8192

TBG = 256
T = B // TBG
WMAX = T + E - 1
CHUNK = 8
CW = SEQ // CHUNK
NR = B * CHUNK
SC_WIN = 128
RW = SC_WIN // CHUNK


@functools.cache
def _vector_mesh():
    return plsc.VectorSubcoreMesh(
        core_axis_name="core", subcore_axis_name="subcore")


TBC = 256
NC = B // TBC


def _routing_kernel(ids_ref, dest_ref, off_ref, oh_ref, cs_ref, tot_ref):
    ids = ids_ref[...]
    lane = jax.lax.broadcasted_iota(jnp.int32, (B, E), 1)
    oh = (ids == lane).astype(jnp.float32)
    oh_ref[...] = oh
    r = jax.lax.broadcasted_iota(jnp.int32, (TBC, TBC), 0)
    c = jax.lax.broadcasted_iota(jnp.int32, (TBC, TBC), 1)
    ltri = (c <= r).astype(jnp.float32)

    for ci in range(NC):
        chunk = oh_ref[pl.ds(ci * TBC, TBC), :]
        cs = jnp.dot(ltri, chunk, preferred_element_type=jnp.float32)
        cs_ref[pl.ds(ci * TBC, TBC), :] = cs
        tot_ref[ci, :] = cs[TBC - 1, :]

    rc = jax.lax.broadcasted_iota(jnp.int32, (NC, NC), 0)
    cc = jax.lax.broadcasted_iota(jnp.int32, (NC, NC), 1)
    strict = (cc < rc).astype(jnp.float32)
    tot = tot_ref[...]
    carry = jnp.dot(strict, tot, preferred_element_type=jnp.float32)
    counts = jnp.sum(tot, axis=0, keepdims=True)

    rowc = jax.lax.broadcasted_iota(jnp.int32, (B, NC), 0) // TBC
    sel = (rowc == jax.lax.broadcasted_iota(jnp.int32, (B, NC), 1)
           ).astype(jnp.float32)
    carry_rows = jnp.dot(sel, carry, preferred_element_type=jnp.float32,
                         precision=jax.lax.Precision.HIGHEST)
    csum = cs_ref[...] + carry_rows
    rank = jnp.sum(oh * csum, axis=1, keepdims=True) - 1.0
    re = jax.lax.broadcasted_iota(jnp.int32, (E, E), 0)
    ce = jax.lax.broadcasted_iota(jnp.int32, (E, E), 1)
    m = (re < ce).astype(jnp.float32)
    offs = jnp.dot(counts, m, preferred_element_type=jnp.float32,
                   precision=jax.lax.Precision.HIGHEST)
    dest = rank + jnp.sum(oh * offs, axis=1, keepdims=True)
    dest_ref[...] = dest.astype(jnp.int32)
    off_ref[...] = offs.astype(jnp.int32)


def _routing(ids):
    return pl.pallas_call(
        _routing_kernel,
        out_shape=(jax.ShapeDtypeStruct((B, 1), jnp.int32),
                   jax.ShapeDtypeStruct((1, E), jnp.int32)),
        scratch_shapes=[pltpu.VMEM((B, E), jnp.float32),
                        pltpu.VMEM((B, E), jnp.float32),
                        pltpu.VMEM((NC, E), jnp.float32)],
    )(ids)


def _dispatch(x, idx):
    @functools.partial(
        pl.kernel,
        out_type=jax.ShapeDtypeStruct((NR, CW), jnp.float32),
        mesh=_vector_mesh())
    def run(x_hbm, i_hbm, o_hbm):
        def body(x_vmem, i_vmem):
            for k in range(CHUNK):
                pltpu.sync_copy(
                    x_vmem.at[:, pl.ds(k * CW, CW)],
                    o_hbm.at[i_vmem.at[0, pl.ds(k * RW, RW)]])

        pltpu.emit_pipeline(
            body,
            grid=(B // RW,),
            in_specs=[pl.BlockSpec((RW, SEQ + 1), lambda i: (i, 0)),
                      pl.BlockSpec((1, SC_WIN), lambda i: (0, i))],
            out_specs=[],
            core_axis_name=("core", "subcore"),
            dimension_semantics=(pltpu.PARALLEL,),
        )(x_hbm, i_hbm)

    return run(x, idx)


def _combine(ys4, idx):
    @functools.partial(
        pl.kernel,
        out_type=jax.ShapeDtypeStruct((B, SEQ), jnp.float32),
        mesh=_vector_mesh())
    def run(ys_hbm, i_hbm, o_hbm):
        def body(i_vmem, o_vmem):
            for k in range(CHUNK):
                pltpu.sync_copy(
                    ys_hbm.at[i_vmem.at[0, pl.ds(k * RW, RW)]],
                    o_vmem.at[:, pl.ds(k * CW, CW)])

        pltpu.emit_pipeline(
            body,
            grid=(B // RW,),
            in_specs=[pl.BlockSpec((1, SC_WIN), lambda i: (0, i))],
            out_specs=[pl.BlockSpec((RW, SEQ), lambda i: (i, 0))],
            core_axis_name=("core", "subcore"),
            dimension_semantics=(pltpu.PARALLEL,),
        )(i_hbm, o_hbm)

    return run(ys4, idx)


def _metadata(offsets):
    o = offsets
    ends = jnp.concatenate([o[1:], jnp.full((1,), B, jnp.int32)])
    counts = ends - o
    f = o // TBG
    l = (ends - 1) // TBG
    tpg = jnp.where(counts > 0, l - f + 1, 0)
    cw = jnp.concatenate([jnp.zeros((1,), jnp.int32),
                          jnp.cumsum(tpg).astype(jnp.int32)])
    total = cw[E]
    w = jnp.arange(WMAX, dtype=jnp.int32)
    gid = jnp.sum((w[:, None] >= cw[None, 1:]).astype(jnp.int32), axis=1)
    gid = jnp.minimum(gid, E - 1)
    tile = f[gid] + (w - cw[gid])
    valid = w < total
    tile = jnp.where(valid, tile, T - 1)
    start = jnp.where(valid, jnp.maximum(o[gid], tile * TBG), 0)
    end = jnp.where(valid, jnp.minimum(ends[gid], (tile + 1) * TBG), 0)
    first = (start == tile * TBG).astype(jnp.int32)
    return tile, gid, start, end, first


def _mlp_kernel(tl_ref, gd_ref, st_ref, en_ref, fr_ref,
                xs_ref, w1_ref, b1_ref, w2_ref, b2_ref,
                w3_ref, b3_ref, w4_ref, b4_ref, out_ref):
    w = pl.program_id(0)
    start, end, first = st_ref[w], en_ref[w], fr_ref[w]

    @pl.when(start < end)
    def _():
        xv = jnp.concatenate([xs_ref[k] for k in range(CHUNK)], axis=1)
        h = jnp.maximum(
            jnp.dot(xv, w1_ref[0], preferred_element_type=jnp.float32)
            + b1_ref[0], 0.0)
        z = jnp.maximum(
            jnp.dot(h, w2_ref[0], preferred_element_type=jnp.float32)
            + b2_ref[0], 0.0)
        h2 = jnp.maximum(
            jnp.dot(z, w3_ref[0], preferred_element_type=jnp.float32)
            + b3_ref[0], 0.0)
        y = (jnp.dot(h2, w4_ref[0], preferred_element_type=jnp.float32)
             + b4_ref[0])
        rows = (tl_ref[w] * TBG
                + jax.lax.broadcasted_iota(jnp.int32, (TBG, 1), 0))
        m = (rows >= start) & (rows < end)
        for k in range(CHUNK):
            yk = y[:, k * CW:(k + 1) * CW]

            @pl.when(first == 1)
            def _(yk=yk, k=k):
                out_ref[k] = jnp.where(m, yk, 0.0)

            @pl.when(first == 0)
            def _(yk=yk, k=k):
                out_ref[k] = jnp.where(m, yk, out_ref[k])


def _grouped_mlp(xs5, meta, W1, b1, W2, b2, W3, b3, W4, b4):
    tile, gid, start, end, first = meta
    grid_spec = pltpu.PrefetchScalarGridSpec(
        num_scalar_prefetch=5,
        grid=(WMAX,),
        in_specs=[
            pl.BlockSpec((CHUNK, TBG, CW),
                         lambda w, tl, gd, st, en, fr: (0, tl[w], 0)),
            pl.BlockSpec((1, SEQ, HID),
                         lambda w, tl, gd, st, en, fr: (gd[w], 0, 0)),
            pl.BlockSpec((1, 1, HID),
                         lambda w, tl, gd, st, en, fr: (gd[w], 0, 0)),
            pl.BlockSpec((1, HID, ENC),
                         lambda w, tl, gd, st, en, fr: (gd[w], 0, 0)),
            pl.BlockSpec((1, 1, ENC),
                         lambda w, tl, gd, st, en, fr: (gd[w], 0, 0)),
            pl.BlockSpec((1, ENC, HID),
                         lambda w, tl, gd, st, en, fr: (gd[w], 0, 0)),
            pl.BlockSpec((1, 1, HID),
                         lambda w, tl, gd, st, en, fr: (gd[w], 0, 0)),
            pl.BlockSpec((1, HID, SEQ),
                         lambda w, tl, gd, st, en, fr: (gd[w], 0, 0)),
            pl.BlockSpec((1, 1, SEQ),
                         lambda w, tl, gd, st, en, fr: (gd[w], 0, 0)),
        ],
        out_specs=pl.BlockSpec((CHUNK, TBG, CW),
                               lambda w, tl, gd, st, en, fr: (0, tl[w], 0)),
    )
    return pl.pallas_call(
        _mlp_kernel,
        grid_spec=grid_spec,
        out_shape=jax.ShapeDtypeStruct((CHUNK, B, CW), jnp.float32),
    )(tile, gid, start, end, first, xs5,
      W1, b1.reshape(E, 1, HID), W2, b2.reshape(E, 1, ENC),
      W3, b3.reshape(E, 1, HID), W4, b4.reshape(E, 1, SEQ))


@jax.jit
def kernel(x, W1, b1, W2, b2, W3, b3, W4, b4):
    ids = x[:, SEQ].astype(jnp.int32).reshape(B, 1)
    dest, off = _routing(ids)
    dest_r = dest.reshape(B // RW, RW)
    idx = (dest_r[:, None, :]
           + (B * jnp.arange(CHUNK, dtype=jnp.int32))[None, :, None])
    idx = idx.reshape(1, NR)
    meta = _metadata(off[0])
    xs4 = _dispatch(x, idx)
    xs5 = xs4.reshape(CHUNK, B, CW)
    ys5 = _grouped_mlp(xs5, meta, W1, b1, W2, b2, W3, b3, W4, b4)
    ys4 = ys5.reshape(NR, CW)
    return _combine(ys4, idx)
